# Initial kernel scaffold; baseline (speedup 1.0000x reference)
import jax
import jax.numpy as jnp
from jax.experimental import pallas as pl


def kernel(z, codebook_0, W_in_0, b_in_0, W_out_0, b_out_0, codebook_1, W_in_1, b_in_1, W_out_1, b_out_1, codebook_2, W_in_2, b_in_2, W_out_2, b_out_2, codebook_3, W_in_3, b_in_3, W_out_3, b_out_3):
    raise NotImplementedError("write your pallas kernel here")


# speedup vs baseline: 1.4975x; 1.4975x over previous
import functools

import jax
import jax.numpy as jnp
from jax import lax
from jax.experimental import pallas as pl
from jax.experimental.pallas import tpu as pltpu
from jax.experimental.pallas import tpu_sc as plsc

_B, _C, _H, _W = 16, 768, 32, 32
_K, _D, _NQ = 8192, 64, 4
_N = _B * _H * _W
_TS = 256
_KC = 2048
_TD = 2048
_DP = 128
_NC, _NS = 2, 16
_NW = _NC * _NS
_BPW = _N // _NW
_ICH = _BPW // 128

_F32 = jnp.float32


def _stage_body(res_ref, wi_ref, bi_ref, cbt_ref, nh_ref, idx_ref, loss_ref):
    t = pl.program_id(0)
    ze = lax.dot_general(res_ref[...], wi_ref[...], (((1,), (1,)), ((), ())),
                         preferred_element_type=_F32) + bi_ref[...]
    run_min = None
    run_idx = None
    for c in range(_K // _KC):
        pc = jnp.dot(ze, cbt_ref[:, c * _KC:(c + 1) * _KC],
                     preferred_element_type=_F32)
        sc = nh_ref[:, c * _KC:(c + 1) * _KC] - pc
        cmin = jnp.min(sc, axis=1, keepdims=True)
        kio = lax.broadcasted_iota(jnp.int32, (_TS, _KC), 1) + c * _KC
        cidx = jnp.min(jnp.where(sc == cmin, kio, _K),
                       axis=1, keepdims=True)
        if run_min is None:
            run_min, run_idx = cmin, cidx
        else:
            upd = cmin < run_min
            run_idx = jnp.where(upd, cidx, run_idx)
            run_min = jnp.minimum(run_min, cmin)

    t1 = jnp.sum(ze * ze, axis=1, keepdims=True)
    part = jnp.sum(t1 + 2.0 * run_min)

    idx_ref[...] = run_idx.reshape(1, 1, _TS)

    @pl.when(t == 0)
    def _():
        loss_ref[...] = jnp.zeros_like(loss_ref)
    loss_ref[...] = loss_ref[...] + jnp.reshape(part, (1, 1))


def _stage(res, wi, bi, cbt, nh):
    nt = _N // _TS
    return pl.pallas_call(
        _stage_body,
        grid=(nt,),
        in_specs=[pl.BlockSpec((_TS, _C), lambda t: (t, 0)),
                  pl.BlockSpec((_D, _C), lambda t: (0, 0)),
                  pl.BlockSpec((1, _D), lambda t: (0, 0)),
                  pl.BlockSpec((_D, _K), lambda t: (0, 0)),
                  pl.BlockSpec((1, _K), lambda t: (0, 0))],
        out_specs=[pl.BlockSpec((1, 1, _TS), lambda t: (t, 0, 0)),
                   pl.BlockSpec((1, 1), lambda t: (0, 0))],
        out_shape=[jax.ShapeDtypeStruct((nt, 1, _TS), jnp.int32),
                   jax.ShapeDtypeStruct((1, 1), _F32)],
    )(res, wi, bi, cbt, nh)


@functools.lru_cache(maxsize=1)
def _get_sc_gather():
    mesh = plsc.VectorSubcoreMesh(core_axis_name="c", subcore_axis_name="s")

    @functools.partial(
        pl.kernel,
        mesh=mesh,
        out_type=jax.ShapeDtypeStruct((_N, _DP), _F32),
        scratch_types=[pltpu.VMEM((_ICH, 128), jnp.int32),
                       pltpu.VMEM((_BPW, _DP), _F32),
                       pltpu.SemaphoreType.DMA],
    )
    def _sc_gather(table_hbm, idx_hbm, out_hbm, idx_v, rows_v, sem):
        wid = lax.axis_index("s") * _NC + lax.axis_index("c")
        pltpu.sync_copy(idx_hbm.at[wid], idx_v)
        cps = [pltpu.async_copy(table_hbm.at[idx_v.at[j]],
                                rows_v.at[pl.ds(j * 128, 128)], sem)
               for j in range(_ICH)]
        for cp in cps:
            cp.wait()
        pltpu.sync_copy(rows_v, out_hbm.at[pl.ds(wid * _BPW, _BPW)])

    return _sc_gather


def _update_body(res_ref, zq_ref, wo_ref, bo_ref, out_ref):
    out = lax.dot_general(zq_ref[...], wo_ref[...], (((1,), (1,)), ((), ())),
                          preferred_element_type=_F32) + bo_ref[...]
    out_ref[...] = res_ref[...] - out


def _update(res, zq, wo_pad, bo):
    return pl.pallas_call(
        _update_body,
        grid=(_N // _TD,),
        in_specs=[pl.BlockSpec((_TD, _C), lambda t: (t, 0)),
                  pl.BlockSpec((_TD, _DP), lambda t: (t, 0)),
                  pl.BlockSpec((_C, _DP), lambda t: (0, 0)),
                  pl.BlockSpec((1, _C), lambda t: (0, 0))],
        out_specs=pl.BlockSpec((_TD, _C), lambda t: (t, 0)),
        out_shape=jax.ShapeDtypeStruct((_N, _C), _F32),
    )(res, zq, wo_pad, bo)


def _final_body(tok_ref, res_ref, zq_ref, wo_ref, bo_ref, out_ref):
    out = lax.dot_general(zq_ref[...], wo_ref[...], (((1,), (1,)), ((), ())),
                          preferred_element_type=_F32) + bo_ref[...]
    res4 = res_ref[...] - out
    out_ref[...] = tok_ref[...] - res4


def _final(tokens, res, zq, wo_pad, bo):
    return pl.pallas_call(
        _final_body,
        grid=(_N // _TD,),
        in_specs=[pl.BlockSpec((_TD, _C), lambda t: (t, 0)),
                  pl.BlockSpec((_TD, _C), lambda t: (t, 0)),
                  pl.BlockSpec((_TD, _DP), lambda t: (t, 0)),
                  pl.BlockSpec((_C, _DP), lambda t: (0, 0)),
                  pl.BlockSpec((1, _C), lambda t: (0, 0))],
        out_specs=pl.BlockSpec((_TD, _C), lambda t: (t, 0)),
        out_shape=jax.ShapeDtypeStruct((_N, _C), _F32),
    )(tokens, res, zq, wo_pad, bo)


def kernel(z,
           codebook_0, W_in_0, b_in_0, W_out_0, b_out_0,
           codebook_1, W_in_1, b_in_1, W_out_1, b_out_1,
           codebook_2, W_in_2, b_in_2, W_out_2, b_out_2,
           codebook_3, W_in_3, b_in_3, W_out_3, b_out_3):
    cbs = [codebook_0, codebook_1, codebook_2, codebook_3]
    wis = [W_in_0, W_in_1, W_in_2, W_in_3]
    bis = [b_in_0, b_in_1, b_in_2, b_in_3]
    wos = [W_out_0, W_out_1, W_out_2, W_out_3]
    bos = [b_out_0, b_out_1, b_out_2, b_out_3]

    tokens = z.transpose(0, 2, 3, 1).reshape(_N, _C)

    cbts = [cb.T for cb in cbs]
    nhs = [(0.5 * jnp.sum(cb * cb, axis=1)).reshape(1, _K) for cb in cbs]
    cb_pads = [jnp.concatenate([cb, jnp.zeros((_K, _DP - _D), _F32)], axis=1)
               for cb in cbs]
    wo_pads = [jnp.concatenate([wo, jnp.zeros((_C, _DP - _D), _F32)], axis=1)
               for wo in wos]

    sc_gather = _get_sc_gather()
    res = tokens
    idxs, lparts = [], []
    for i in range(_NQ):
        idx3, lp = _stage(res, wis[i], bis[i].reshape(1, _D), cbts[i], nhs[i])
        idx = idx3.reshape(_N)
        zq = sc_gather(cb_pads[i], idx.reshape(_NW, _ICH, 128))
        idxs.append(idx)
        lparts.append(lp[0, 0])
        if i < _NQ - 1:
            res = _update(res, zq, wo_pads[i], bos[i].reshape(1, _C))
        else:
            out_tok = _final(tokens, res, zq, wo_pads[i], bos[i].reshape(1, _C))

    z_q_total = out_tok.reshape(_B, _H, _W, _C).transpose(0, 3, 1, 2)
    codes = [ix.reshape(_B, _H, _W) for ix in idxs]
    loss = (1.25 / (_N * _D * _NQ)) * (lparts[0] + lparts[1]
                                       + lparts[2] + lparts[3])
    return (z_q_total, codes[0], codes[1], codes[2], codes[3], loss)
